# baseline (speedup 1.0000x reference)
import functools

import jax
import jax.numpy as jnp
from jax import lax
from jax.experimental import pallas as pl
from jax.experimental.pallas import tpu as pltpu
from jax.experimental.pallas import tpu_sc as plsc

_B = 4096
_H = 768
_V = 512
_E = 8
_C = 3 * _E
_BT = 1024
_NW = 32
_CHUNK = _B // _NW
_L = 16


def _tc_body(t_ref, v_ref, tW1_ref, tb1_ref, tW2_ref, tb2_ref,
             vW1_ref, vb1_ref, vW2_ref, vb2_ref,
             eW1_ref, eb1_ref, eW2_ref, eb2_ref, eW3_ref, eb3_ref,
             out_ref,
             tW1s, tW2s, vW1s, vW2s, eW1s, eW2s, eW3s):
    f32 = jnp.float32
    bf = jnp.bfloat16

    @pl.when(pl.program_id(0) == 0)
    def _cast_weights():
        tW1s[...] = tW1_ref[...].astype(bf)
        tW2s[...] = tW2_ref[...].astype(bf)
        vW1s[...] = vW1_ref[...].astype(bf)
        vW2s[...] = vW2_ref[...].astype(bf)
        eW1s[...] = eW1_ref[...].astype(bf)
        eW2s[...] = eW2_ref[...].astype(bf)
        eW3s[...] = eW3_ref[...].astype(bf)

    tb = t_ref[...].astype(bf)
    vb = v_ref[...].astype(bf)
    cb = jnp.concatenate([tb, vb], axis=1)
    rt = jnp.maximum(
        jnp.dot(tb, tW1s[...], preferred_element_type=f32).astype(bf)
        + tb1_ref[...].astype(bf), 0)
    tlog = (jnp.dot(rt, tW2s[...], preferred_element_type=f32)
            + tb2_ref[...])
    rv = jnp.maximum(
        jnp.dot(vb, vW1s[...], preferred_element_type=f32).astype(bf)
        + vb1_ref[...].astype(bf), 0)
    vlog = (jnp.dot(rv, vW2s[...], preferred_element_type=f32)
            + vb2_ref[...])
    outs = []
    for e in range(_E):
        h1 = jnp.maximum(
            jnp.dot(cb, eW1s[e], preferred_element_type=f32).astype(bf)
            + eb1_ref[e].astype(bf), 0)
        h2 = jnp.maximum(
            jnp.dot(h1, eW2s[e], preferred_element_type=f32).astype(bf)
            + eb2_ref[e].astype(bf), 0)
        o = (jnp.dot(h2, eW3s[e], preferred_element_type=f32)
             + eb3_ref[e])
        outs.append(jax.nn.sigmoid(o))
    out_ref[...] = jnp.concatenate([tlog, vlog] + outs, axis=1)


def _tc_dense(text, video, tW1, tb1, tW2, tb2, vW1, vb1, vW2, vb2,
              eW1, eb1, eW2, eb2, eW3, eb3, nb=_B, off=0, interpret=False):
    grid = nb // _BT
    step0 = off // _BT
    bf = jnp.bfloat16
    full = lambda *shape: pl.BlockSpec(shape, lambda i: (0,) * len(shape))
    row = lambda cols: pl.BlockSpec((_BT, cols), lambda i: (i + step0, 0))
    return pl.pallas_call(
        _tc_body,
        grid=(grid,),
        in_specs=[
            row(_H), row(_V),
            full(_H, 256), full(256), full(256, _E), full(_E),
            full(_V, 256), full(256), full(256, _E), full(_E),
            full(_E, _H + _V, 256), full(_E, 256),
            full(_E, 256, 128), full(_E, 128),
            full(_E, 128, 1), full(_E, 1),
        ],
        out_specs=[pl.BlockSpec((_BT, _C), lambda i: (i, 0))],
        out_shape=[jax.ShapeDtypeStruct((nb, _C), jnp.float32)],
        scratch_shapes=[
            pltpu.VMEM((_H, 256), bf), pltpu.VMEM((256, _E), bf),
            pltpu.VMEM((_V, 256), bf), pltpu.VMEM((256, _E), bf),
            pltpu.VMEM((_E, _H + _V, 256), bf), pltpu.VMEM((_E, 256, 128), bf),
            pltpu.VMEM((_E, 128, 1), bf),
        ],
        compiler_params=pltpu.CompilerParams(
            dimension_semantics=("arbitrary",)),
        interpret=interpret,
    )(text, video, tW1, tb1, tW2, tb2, vW1, vb1, vW2, vb2,
      eW1, eb1, eW2, eb2, eW3, eb3)[0]


def _top2_weighted(buf_v, tok24, log_off):
    neg = jnp.full((_L,), -jnp.inf, jnp.float32)
    zero_i = jnp.zeros((_L,), jnp.int32)
    m1, i1, m2, i2 = neg, zero_i, neg, zero_i
    for e in range(_E):
        ei = jnp.full((_L,), e, jnp.int32)
        val = plsc.load_gather(buf_v, [tok24 + (log_off + e)])
        gt1 = val > m1
        gt2 = val > m2
        m2 = jnp.where(gt1, m1, jnp.where(gt2, val, m2))
        i2 = jnp.where(gt1, i1, jnp.where(gt2, ei, i2))
        m1 = jnp.where(gt1, val, m1)
        i1 = jnp.where(gt1, ei, i1)
    d = jnp.exp(m2 - m1)
    s = d + 1.0
    w1 = 1.0 / s
    w2 = d / s
    eo_base = tok24 + (2 * _E)
    g1 = plsc.load_gather(buf_v, [eo_base + i1])
    g2 = plsc.load_gather(buf_v, [eo_base + i2])
    return w1 * g1 + w2 * g2


def _sc_route_fn(nb=_B):
    chunk = nb // _NW
    mesh = plsc.VectorSubcoreMesh(core_axis_name="c", subcore_axis_name="s")

    @functools.partial(
        pl.kernel, mesh=mesh,
        out_type=[jax.ShapeDtypeStruct((nb,), jnp.float32),
                  jax.ShapeDtypeStruct((nb,), jnp.float32)],
        scratch_types=[
            pltpu.VMEM((chunk * _C,), jnp.float32),
            pltpu.VMEM((chunk,), jnp.float32),
            pltpu.VMEM((chunk,), jnp.float32),
            pltpu.SemaphoreType.DMA,
        ],
        compiler_params=pltpu.CompilerParams(needs_layout_passes=False),
    )
    def sc_route(comb_hbm, tout_hbm, vout_hbm, buf_v, tc_v, vc_v, sem):
        wid = lax.axis_index("s") * 2 + lax.axis_index("c")
        base = wid * chunk
        cp = pltpu.async_copy(
            comb_hbm.at[pl.ds(base * _C, chunk * _C)], buf_v, sem)
        cp.wait()
        for g in range(chunk // _L):
            tok24 = (lax.iota(jnp.int32, _L) + (g * _L)) * _C
            tc_v[pl.ds(g * _L, _L)] = _top2_weighted(buf_v, tok24, 0)
            vc_v[pl.ds(g * _L, _L)] = _top2_weighted(buf_v, tok24, _E)
        pltpu.sync_copy(tc_v, tout_hbm.at[pl.ds(base, chunk)])
        pltpu.sync_copy(vc_v, vout_hbm.at[pl.ds(base, chunk)])

    return sc_route


def kernel(text_features, video_features, tW1, tb1, tW2, tb2,
           vW1, vb1, vW2, vb2, eW1, eb1, eW2, eb2, eW3, eb3):
    half = _B // 2
    args = (tW1, tb1, tW2, tb2, vW1, vb1, vW2, vb2,
            eW1, eb1, eW2, eb2, eW3, eb3)
    sc = _sc_route_fn(half)
    comb0 = _tc_dense(text_features, video_features, *args, nb=half, off=0)
    comb1 = _tc_dense(text_features, video_features, *args, nb=half, off=half)
    tc0, vc0 = sc(comb0.reshape(-1))
    tc1, vc1 = sc(comb1.reshape(-1))
    tconf = jnp.concatenate([tc0, tc1])
    vconf = jnp.concatenate([vc0, vc1])
    return tconf.reshape(_B, 1), vconf.reshape(_B, 1)


# speedup vs baseline: 1.0736x; 1.0736x over previous
import functools

import jax
import jax.numpy as jnp
from jax import lax
from jax.experimental import pallas as pl
from jax.experimental.pallas import tpu as pltpu
from jax.experimental.pallas import tpu_sc as plsc

_B = 4096
_H = 768
_V = 512
_D = _H + _V
_E = 8
_C = 3 * _E
_BT = 1024
_NW = 32
_L = 16


def _tc_body(t_ref, v_ref, tW1_ref, tb1_ref, tW2_ref, tb2_ref,
             vW1_ref, vb1_ref, vW2_ref, vb2_ref,
             eW1_hbm, eb1_ref, eW2_ref, eb2_ref, eW3_ref, eb3_ref,
             out_ref,
             tW1s, tW2s, vW1s, vW2s, eW1s, eW2s, eW3s, w1f, sems):
    f32 = jnp.float32
    bf = jnp.bfloat16
    first = pl.program_id(0) == 0

    @pl.when(first)
    def _start():
        pltpu.make_async_copy(eW1_hbm.at[0], w1f.at[0], sems.at[0]).start()
        tW1s[...] = tW1_ref[...].astype(bf)
        tW2s[...] = tW2_ref[...].astype(bf)
        vW1s[...] = vW1_ref[...].astype(bf)
        vW2s[...] = vW2_ref[...].astype(bf)
        eW2s[...] = eW2_ref[...].astype(bf)
        eW3s[...] = eW3_ref[...].astype(bf)

    tb = t_ref[...].astype(bf)
    vb = v_ref[...].astype(bf)
    cb = jnp.concatenate([tb, vb], axis=1)
    rt = jnp.maximum(
        jnp.dot(tb, tW1s[...], preferred_element_type=f32).astype(bf)
        + tb1_ref[...].astype(bf), 0)
    tlog = (jnp.dot(rt, tW2s[...], preferred_element_type=f32)
            + tb2_ref[...])
    rv = jnp.maximum(
        jnp.dot(vb, vW1s[...], preferred_element_type=f32).astype(bf)
        + vb1_ref[...].astype(bf), 0)
    vlog = (jnp.dot(rv, vW2s[...], preferred_element_type=f32)
            + vb2_ref[...])
    outs = []
    for e in range(_E):
        @pl.when(first)
        def _pipe(e=e):
            if e + 1 < _E:
                pltpu.make_async_copy(
                    eW1_hbm.at[e + 1], w1f.at[(e + 1) % 2],
                    sems.at[(e + 1) % 2]).start()
            pltpu.make_async_copy(
                eW1_hbm.at[e], w1f.at[e % 2], sems.at[e % 2]).wait()
            eW1s[e] = w1f[e % 2].astype(bf)

        h1 = jnp.maximum(
            jnp.dot(cb, eW1s[e], preferred_element_type=f32).astype(bf)
            + eb1_ref[e].astype(bf), 0)
        h2 = jnp.maximum(
            jnp.dot(h1, eW2s[e], preferred_element_type=f32).astype(bf)
            + eb2_ref[e].astype(bf), 0)
        o = (jnp.dot(h2, eW3s[e], preferred_element_type=f32)
             + eb3_ref[e])
        outs.append(jax.nn.sigmoid(o))
    out_ref[...] = jnp.concatenate([tlog, vlog] + outs, axis=1)


def _tc_dense(text, video, tW1, tb1, tW2, tb2, vW1, vb1, vW2, vb2,
              eW1, eb1, eW2, eb2, eW3, eb3, interpret=False):
    grid = _B // _BT
    bf = jnp.bfloat16
    full = lambda *shape: pl.BlockSpec(shape, lambda i: (0,) * len(shape))
    row = lambda cols: pl.BlockSpec((_BT, cols), lambda i: (i, 0))
    return pl.pallas_call(
        _tc_body,
        grid=(grid,),
        in_specs=[
            row(_H), row(_V),
            full(_H, 256), full(256), full(256, _E), full(_E),
            full(_V, 256), full(256), full(256, _E), full(_E),
            pl.BlockSpec(memory_space=pl.ANY), full(_E, 256),
            full(_E, 256, 128), full(_E, 128),
            full(_E, 128, 1), full(_E, 1),
        ],
        out_specs=[pl.BlockSpec((_BT, _C), lambda i: (i, 0))],
        out_shape=[jax.ShapeDtypeStruct((_B, _C), jnp.float32)],
        scratch_shapes=[
            pltpu.VMEM((_H, 256), bf), pltpu.VMEM((256, _E), bf),
            pltpu.VMEM((_V, 256), bf), pltpu.VMEM((256, _E), bf),
            pltpu.VMEM((_E, _D, 256), bf), pltpu.VMEM((_E, 256, 128), bf),
            pltpu.VMEM((_E, 128, 1), bf),
            pltpu.VMEM((2, _D, 256), jnp.float32),
            pltpu.SemaphoreType.DMA((2,)),
        ],
        compiler_params=pltpu.CompilerParams(
            dimension_semantics=("arbitrary",)),
        interpret=interpret,
    )(text, video, tW1, tb1, tW2, tb2, vW1, vb1, vW2, vb2,
      eW1, eb1, eW2, eb2, eW3, eb3)[0]


def _top2_weighted(buf_v, tok24, log_off):
    neg = jnp.full((_L,), -jnp.inf, jnp.float32)
    zero_i = jnp.zeros((_L,), jnp.int32)
    m1, i1, m2, i2 = neg, zero_i, neg, zero_i
    for e in range(_E):
        ei = jnp.full((_L,), e, jnp.int32)
        val = plsc.load_gather(buf_v, [tok24 + (log_off + e)])
        gt1 = val > m1
        gt2 = val > m2
        m2 = jnp.where(gt1, m1, jnp.where(gt2, val, m2))
        i2 = jnp.where(gt1, i1, jnp.where(gt2, ei, i2))
        m1 = jnp.where(gt1, val, m1)
        i1 = jnp.where(gt1, ei, i1)
    d = jnp.exp(m2 - m1)
    s = d + 1.0
    w1 = 1.0 / s
    w2 = d / s
    eo_base = tok24 + (2 * _E)
    g1 = plsc.load_gather(buf_v, [eo_base + i1])
    g2 = plsc.load_gather(buf_v, [eo_base + i2])
    return w1 * g1 + w2 * g2


def _sc_route_fn(nb=_B):
    chunk = nb // _NW
    mesh = plsc.VectorSubcoreMesh(core_axis_name="c", subcore_axis_name="s")

    @functools.partial(
        pl.kernel, mesh=mesh,
        out_type=[jax.ShapeDtypeStruct((nb,), jnp.float32),
                  jax.ShapeDtypeStruct((nb,), jnp.float32)],
        scratch_types=[
            pltpu.VMEM((chunk * _C,), jnp.float32),
            pltpu.VMEM((chunk,), jnp.float32),
            pltpu.VMEM((chunk,), jnp.float32),
            pltpu.SemaphoreType.DMA,
        ],
        compiler_params=pltpu.CompilerParams(needs_layout_passes=False),
    )
    def sc_route(comb_hbm, tout_hbm, vout_hbm, buf_v, tc_v, vc_v, sem):
        wid = lax.axis_index("s") * 2 + lax.axis_index("c")
        base = wid * chunk
        cp = pltpu.async_copy(
            comb_hbm.at[pl.ds(base * _C, chunk * _C)], buf_v, sem)
        cp.wait()
        for g in range(chunk // _L):
            tok24 = (lax.iota(jnp.int32, _L) + (g * _L)) * _C
            tc_v[pl.ds(g * _L, _L)] = _top2_weighted(buf_v, tok24, 0)
            vc_v[pl.ds(g * _L, _L)] = _top2_weighted(buf_v, tok24, _E)
        pltpu.sync_copy(tc_v, tout_hbm.at[pl.ds(base, chunk)])
        pltpu.sync_copy(vc_v, vout_hbm.at[pl.ds(base, chunk)])

    return sc_route


def kernel(text_features, video_features, tW1, tb1, tW2, tb2,
           vW1, vb1, vW2, vb2, eW1, eb1, eW2, eb2, eW3, eb3):
    comb = _tc_dense(
        text_features, video_features, tW1, tb1, tW2, tb2,
        vW1, vb1, vW2, vb2, eW1, eb1, eW2, eb2, eW3, eb3)
    tconf, vconf = _sc_route_fn()(comb.reshape(-1))
    return tconf.reshape(_B, 1), vconf.reshape(_B, 1)


# speedup vs baseline: 1.1133x; 1.0369x over previous
import functools

import jax
import jax.numpy as jnp
from jax import lax
from jax.experimental import pallas as pl
from jax.experimental.pallas import tpu as pltpu
from jax.experimental.pallas import tpu_sc as plsc

_B = 4096
_H = 768
_V = 512
_D = _H + _V
_E = 8
_C = 3 * _E
_BT = 1024
_NW = 32
_L = 16


def _tc_body(t_ref, v_ref, tW1_ref, tb1_ref, tW2_ref, tb2_ref,
             vW1_ref, vb1_ref, vW2_ref, vb2_ref,
             eW1_ref, eb1_ref, eW2_ref, eb2_ref, eW3_ref, eb3_ref,
             out_ref,
             tW1s, tW2s, vW1s, vW2s, eW1s, eW2s, eW3s):
    f32 = jnp.float32
    bf = jnp.bfloat16
    first = pl.program_id(0) == 0

    @pl.when(first)
    def _start():
        tW1s[...] = tW1_ref[...].astype(bf)
        tW2s[...] = tW2_ref[...].astype(bf)
        vW1s[...] = vW1_ref[...].astype(bf)
        vW2s[...] = vW2_ref[...].astype(bf)
        eW1s[...] = eW1_ref[...].astype(bf)
        eW2s[...] = eW2_ref[...].astype(bf)
        eW3s[...] = eW3_ref[...].astype(bf)

    tb = t_ref[...].astype(bf)
    vb = v_ref[...].astype(bf)
    cb = jnp.concatenate([tb, vb], axis=1)
    rt = jnp.maximum(
        jnp.dot(tb, tW1s[...], preferred_element_type=f32).astype(bf)
        + tb1_ref[...].astype(bf), 0)
    tlog = (jnp.dot(rt, tW2s[...], preferred_element_type=f32)
            + tb2_ref[...])
    rv = jnp.maximum(
        jnp.dot(vb, vW1s[...], preferred_element_type=f32).astype(bf)
        + vb1_ref[...].astype(bf), 0)
    vlog = (jnp.dot(rv, vW2s[...], preferred_element_type=f32)
            + vb2_ref[...])
    outs = []
    for e in range(_E):
        h1 = jnp.maximum(
            jnp.dot(cb, eW1s[e], preferred_element_type=f32).astype(bf)
            + eb1_ref[e].astype(bf), 0)
        h2 = jnp.maximum(
            jnp.dot(h1, eW2s[e], preferred_element_type=f32).astype(bf)
            + eb2_ref[e].astype(bf), 0)
        o = (jnp.dot(h2, eW3s[e], preferred_element_type=f32)
             + eb3_ref[e])
        outs.append(jax.nn.sigmoid(o))
    out_ref[...] = jnp.concatenate([tlog, vlog] + outs, axis=1)


def _tc_dense(text, video, tW1, tb1, tW2, tb2, vW1, vb1, vW2, vb2,
              eW1, eb1, eW2, eb2, eW3, eb3, interpret=False):
    grid = _B // _BT
    bf = jnp.bfloat16
    full = lambda *shape: pl.BlockSpec(shape, lambda i: (0,) * len(shape))
    row = lambda cols: pl.BlockSpec((_BT, cols), lambda i: (i, 0))
    return pl.pallas_call(
        _tc_body,
        grid=(grid,),
        in_specs=[
            row(_H), row(_V),
            full(_H, 256), full(256), full(256, _E), full(_E),
            full(_V, 256), full(256), full(256, _E), full(_E),
            full(_E, _D, 256), full(_E, 256),
            full(_E, 256, 128), full(_E, 128),
            full(_E, 128, 1), full(_E, 1),
        ],
        out_specs=[pl.BlockSpec((_BT, _C), lambda i: (i, 0))],
        out_shape=[jax.ShapeDtypeStruct((_B, _C), jnp.float32)],
        scratch_shapes=[
            pltpu.VMEM((_H, 256), bf), pltpu.VMEM((256, _E), bf),
            pltpu.VMEM((_V, 256), bf), pltpu.VMEM((256, _E), bf),
            pltpu.VMEM((_E, _D, 256), bf), pltpu.VMEM((_E, 256, 128), bf),
            pltpu.VMEM((_E, 128, 1), bf),
        ],
        compiler_params=pltpu.CompilerParams(
            dimension_semantics=("arbitrary",)),
        interpret=interpret,
    )(text, video, tW1, tb1, tW2, tb2, vW1, vb1, vW2, vb2,
      eW1, eb1, eW2, eb2, eW3, eb3)[0]


def _top2_weighted(buf_v, tok24, log_off):
    neg = jnp.full((_L,), -jnp.inf, jnp.float32)
    zero_i = jnp.zeros((_L,), jnp.int32)
    m1, i1, m2, i2 = neg, zero_i, neg, zero_i
    for e in range(_E):
        ei = jnp.full((_L,), e, jnp.int32)
        val = plsc.load_gather(buf_v, [tok24 + (log_off + e)])
        gt1 = val > m1
        gt2 = val > m2
        m2 = jnp.where(gt1, m1, jnp.where(gt2, val, m2))
        i2 = jnp.where(gt1, i1, jnp.where(gt2, ei, i2))
        m1 = jnp.where(gt1, val, m1)
        i1 = jnp.where(gt1, ei, i1)
    d = jnp.exp(m2 - m1)
    s = d + 1.0
    w1 = 1.0 / s
    w2 = d / s
    eo_base = tok24 + (2 * _E)
    g1 = plsc.load_gather(buf_v, [eo_base + i1])
    g2 = plsc.load_gather(buf_v, [eo_base + i2])
    return w1 * g1 + w2 * g2


def _sc_route_fn(nb=_B):
    chunk = nb // _NW
    mesh = plsc.VectorSubcoreMesh(core_axis_name="c", subcore_axis_name="s")

    @functools.partial(
        pl.kernel, mesh=mesh,
        out_type=[jax.ShapeDtypeStruct((nb,), jnp.float32),
                  jax.ShapeDtypeStruct((nb,), jnp.float32)],
        scratch_types=[
            pltpu.VMEM((chunk * _C,), jnp.float32),
            pltpu.VMEM((chunk,), jnp.float32),
            pltpu.VMEM((chunk,), jnp.float32),
            pltpu.SemaphoreType.DMA,
        ],
        compiler_params=pltpu.CompilerParams(needs_layout_passes=False),
    )
    def sc_route(comb_hbm, tout_hbm, vout_hbm, buf_v, tc_v, vc_v, sem):
        wid = lax.axis_index("s") * 2 + lax.axis_index("c")
        base = wid * chunk
        cp = pltpu.async_copy(
            comb_hbm.at[pl.ds(base * _C, chunk * _C)], buf_v, sem)
        cp.wait()
        for g in range(chunk // _L):
            tok24 = (lax.iota(jnp.int32, _L) + (g * _L)) * _C
            tc_v[pl.ds(g * _L, _L)] = _top2_weighted(buf_v, tok24, 0)
            vc_v[pl.ds(g * _L, _L)] = _top2_weighted(buf_v, tok24, _E)
        pltpu.sync_copy(tc_v, tout_hbm.at[pl.ds(base, chunk)])
        pltpu.sync_copy(vc_v, vout_hbm.at[pl.ds(base, chunk)])

    return sc_route


def kernel(text_features, video_features, tW1, tb1, tW2, tb2,
           vW1, vb1, vW2, vb2, eW1, eb1, eW2, eb2, eW3, eb3):
    comb = _tc_dense(
        text_features, video_features, tW1, tb1, tW2, tb2,
        vW1, vb1, vW2, vb2, eW1, eb1, eW2, eb2, eW3, eb3)
    tconf, vconf = _sc_route_fn()(comb.reshape(-1))
    return tconf.reshape(_B, 1), vconf.reshape(_B, 1)


# speedup vs baseline: 1.4953x; 1.3432x over previous
import functools

import jax
import jax.numpy as jnp
from jax import lax
from jax.experimental import pallas as pl
from jax.experimental.pallas import tpu as pltpu
from jax.experimental.pallas import tpu_sc as plsc

_B = 4096
_H = 768
_V = 512
_D = _H + _V
_E = 8
_C = 3 * _E
_BT = 1024
_NW = 32
_L = 16


def _tc_body(t_ref, v_ref, tW1_ref, tb1_ref, tW2_ref, tb2_ref,
             vW1_ref, vb1_ref, vW2_ref, vb2_ref,
             eW1_ref, eb1_ref, eW2_ref, eb2_ref, eW3b_ref, eb3r_ref,
             out_ref,
             tW1s, tW2s, vW1s, vW2s, eW1s, eW2s):
    f32 = jnp.float32
    bf = jnp.bfloat16
    first = pl.program_id(0) == 0

    @pl.when(first)
    def _start():
        tW1s[...] = tW1_ref[...].astype(bf)
        tW2s[...] = tW2_ref[...].astype(bf)
        vW1s[...] = vW1_ref[...].astype(bf)
        vW2s[...] = vW2_ref[...].astype(bf)
        for e in range(_E):
            eW1s[:, e * 256:(e + 1) * 256] = eW1_ref[e].astype(bf)
        eW2s[...] = eW2_ref[...].astype(bf)

    tb = t_ref[...].astype(bf)
    vb = v_ref[...].astype(bf)
    cb = jnp.concatenate([tb, vb], axis=1)
    rt_acc = jnp.dot(tb, tW1s[...], preferred_element_type=f32)
    rv_acc = jnp.dot(vb, vW1s[...], preferred_element_type=f32)
    h1_acc = jnp.dot(cb, eW1s[...], preferred_element_type=f32)
    rt = jnp.maximum(rt_acc.astype(bf) + tb1_ref[...].astype(bf), 0)
    rv = jnp.maximum(rv_acc.astype(bf) + vb1_ref[...].astype(bf), 0)
    h1 = jnp.maximum(h1_acc.astype(bf) + eb1_ref[...].astype(bf), 0)
    tlog = (jnp.dot(rt, tW2s[...], preferred_element_type=f32)
            + tb2_ref[...])
    vlog = (jnp.dot(rv, vW2s[...], preferred_element_type=f32)
            + vb2_ref[...])
    h2s = []
    for e in range(_E):
        h2_acc = jnp.dot(h1[:, e * 256:(e + 1) * 256], eW2s[e],
                         preferred_element_type=f32)
        h2s.append(jnp.maximum(h2_acc.astype(bf) + eb2_ref[e].astype(bf), 0))
    h2all = jnp.concatenate(h2s, axis=1)
    oall = (jnp.dot(h2all, eW3b_ref[...], preferred_element_type=f32)
            + eb3r_ref[...])
    eo = jax.nn.sigmoid(oall)
    out_ref[...] = jnp.concatenate([tlog, vlog, eo], axis=1)


def _tc_dense(text, video, tW1, tb1, tW2, tb2, vW1, vb1, vW2, vb2,
              eW1, eb1, eW2, eb2, eW3, eb3, interpret=False):
    grid = _B // _BT
    bf = jnp.bfloat16
    full = lambda *shape: pl.BlockSpec(shape, lambda i: (0,) * len(shape))
    row = lambda cols: pl.BlockSpec((_BT, cols), lambda i: (i, 0))
    return pl.pallas_call(
        _tc_body,
        grid=(grid,),
        in_specs=[
            row(_H), row(_V),
            full(_H, 256), full(256), full(256, _E), full(_E),
            full(_V, 256), full(256), full(256, _E), full(_E),
            full(_E, _D, 256), full(1, _E * 256),
            full(_E, 256, 128), full(_E, 128),
            full(_E * 128, _E), full(1, _E),
        ],
        out_specs=[pl.BlockSpec((_BT, _C), lambda i: (i, 0))],
        out_shape=[jax.ShapeDtypeStruct((_B, _C), jnp.float32)],
        scratch_shapes=[
            pltpu.VMEM((_H, 256), bf), pltpu.VMEM((256, _E), bf),
            pltpu.VMEM((_V, 256), bf), pltpu.VMEM((256, _E), bf),
            pltpu.VMEM((_D, _E * 256), bf), pltpu.VMEM((_E, 256, 128), bf),
        ],
        compiler_params=pltpu.CompilerParams(
            dimension_semantics=("arbitrary",)),
        interpret=interpret,
    )(text, video, tW1, tb1, tW2, tb2, vW1, vb1, vW2, vb2,
      eW1, eb1, eW2, eb2, eW3, eb3)[0]


def _top2_weighted(buf_v, tok24, log_off):
    neg = jnp.full((_L,), -jnp.inf, jnp.float32)
    zero_i = jnp.zeros((_L,), jnp.int32)
    m1, i1, m2, i2 = neg, zero_i, neg, zero_i
    for e in range(_E):
        ei = jnp.full((_L,), e, jnp.int32)
        val = plsc.load_gather(buf_v, [tok24 + (log_off + e)])
        gt1 = val > m1
        gt2 = val > m2
        m2 = jnp.where(gt1, m1, jnp.where(gt2, val, m2))
        i2 = jnp.where(gt1, i1, jnp.where(gt2, ei, i2))
        m1 = jnp.where(gt1, val, m1)
        i1 = jnp.where(gt1, ei, i1)
    d = jnp.exp(m2 - m1)
    s = d + 1.0
    w1 = 1.0 / s
    w2 = d / s
    eo_base = tok24 + (2 * _E)
    g1 = plsc.load_gather(buf_v, [eo_base + i1])
    g2 = plsc.load_gather(buf_v, [eo_base + i2])
    return w1 * g1 + w2 * g2


def _sc_route_fn(nb=_B):
    chunk = nb // _NW
    mesh = plsc.VectorSubcoreMesh(core_axis_name="c", subcore_axis_name="s")

    @functools.partial(
        pl.kernel, mesh=mesh,
        out_type=[jax.ShapeDtypeStruct((nb,), jnp.float32),
                  jax.ShapeDtypeStruct((nb,), jnp.float32)],
        scratch_types=[
            pltpu.VMEM((chunk * _C,), jnp.float32),
            pltpu.VMEM((chunk,), jnp.float32),
            pltpu.VMEM((chunk,), jnp.float32),
            pltpu.SemaphoreType.DMA,
        ],
        compiler_params=pltpu.CompilerParams(needs_layout_passes=False),
    )
    def sc_route(comb_hbm, tout_hbm, vout_hbm, buf_v, tc_v, vc_v, sem):
        wid = lax.axis_index("s") * 2 + lax.axis_index("c")
        base = wid * chunk
        cp = pltpu.async_copy(
            comb_hbm.at[pl.ds(base * _C, chunk * _C)], buf_v, sem)
        cp.wait()
        for g in range(chunk // _L):
            tok24 = (lax.iota(jnp.int32, _L) + (g * _L)) * _C
            tc_v[pl.ds(g * _L, _L)] = _top2_weighted(buf_v, tok24, 0)
            vc_v[pl.ds(g * _L, _L)] = _top2_weighted(buf_v, tok24, _E)
        pltpu.sync_copy(tc_v, tout_hbm.at[pl.ds(base, chunk)])
        pltpu.sync_copy(vc_v, vout_hbm.at[pl.ds(base, chunk)])

    return sc_route


def kernel(text_features, video_features, tW1, tb1, tW2, tb2,
           vW1, vb1, vW2, vb2, eW1, eb1, eW2, eb2, eW3, eb3):
    bf = jnp.bfloat16
    eb1r = eb1.reshape(1, _E * 256)
    eW3b = jax.scipy.linalg.block_diag(*[eW3[e] for e in range(_E)]).astype(bf)
    eb3r = eb3.reshape(1, _E)
    comb = _tc_dense(
        text_features, video_features, tW1, tb1, tW2, tb2,
        vW1, vb1, vW2, vb2, eW1, eb1r, eW2, eb2, eW3b, eb3r)
    tconf, vconf = _sc_route_fn()(comb.reshape(-1))
    return tconf.reshape(_B, 1), vconf.reshape(_B, 1)
